# Initial kernel scaffold; baseline (speedup 1.0000x reference)
import jax
import jax.numpy as jnp
from jax.experimental import pallas as pl


def kernel(outputs, targets):
    raise NotImplementedError("write your pallas kernel here")


# speedup vs baseline: 1.3851x; 1.3851x over previous
import jax
import jax.numpy as jnp
from jax.experimental import pallas as pl
from jax.experimental.pallas import tpu as pltpu

OUT_CH = 257
T = 8192
TBLK = 1024
GRID = T // TBLK


def _loss_body(out_ref, tgt_ref, res_ref, acc_ref):
    i = pl.program_id(0)

    @pl.when(i == 0)
    def _init():
        acc_ref[0] = 0.0
        acc_ref[1] = 0.0
        acc_ref[2] = 0.0

    blk = out_ref[...]
    tgt = tgt_ref[...]
    y = tgt[:, 0:1]
    st = tgt[:, 1:2].astype(jnp.int32)

    col = jax.lax.broadcasted_iota(jnp.int32, (TBLK, OUT_CH), 1)
    is_sym = col >= 1

    neg_inf = jnp.float32(-jnp.inf)
    sym = jnp.where(is_sym, blk, neg_inf)
    m = jnp.max(sym, axis=1, keepdims=True)
    s = jnp.sum(jnp.exp(sym - m), axis=1, keepdims=True)
    logz = m + jnp.log(s)

    ll = jnp.sum(jnp.where(col == st + 1, blk, 0.0), axis=1, keepdims=True)
    ce = logz - ll

    x = blk[:, 0:1]
    bce = jnp.maximum(x, 0.0) - x * y + jnp.log1p(jnp.exp(-jnp.abs(x)))

    acc_ref[0] += jnp.sum(bce)
    acc_ref[1] += jnp.sum(ce * y)
    acc_ref[2] += jnp.sum(y)

    @pl.when(i == GRID - 1)
    def _final():
        count = acc_ref[2]
        symbol_loss = jnp.where(
            count != 0.0, acc_ref[1] / jnp.maximum(count, 1.0), 0.0
        )
        res_ref[0, 0] = acc_ref[0] / T + symbol_loss


def kernel(outputs, targets):
    out2d = outputs.reshape(T, OUT_CH)
    tgt2d = targets.reshape(T, 2)
    res = pl.pallas_call(
        _loss_body,
        grid=(GRID,),
        in_specs=[
            pl.BlockSpec((TBLK, OUT_CH), lambda i: (i, 0)),
            pl.BlockSpec((TBLK, 2), lambda i: (i, 0)),
        ],
        out_specs=pl.BlockSpec((1, 1), lambda i: (0, 0), memory_space=pltpu.SMEM),
        out_shape=jax.ShapeDtypeStruct((1, 1), jnp.float32),
        scratch_shapes=[pltpu.SMEM((3,), jnp.float32)],
        compiler_params=pltpu.CompilerParams(
            dimension_semantics=("arbitrary",),
        ),
    )(out2d, tgt2d)
    return res[0, 0]


# speedup vs baseline: 1.4344x; 1.0356x over previous
import jax
import jax.numpy as jnp
from jax.experimental import pallas as pl
from jax.experimental.pallas import tpu as pltpu

OUT_CH = 257
T = 8192
TBLK = 1024
GRID = T // TBLK


def _loss_body(out_ref, tgt_ref, res_ref, acc_ref):
    i = pl.program_id(0)

    @pl.when(i == 0)
    def _init():
        acc_ref[0] = 0.0
        acc_ref[1] = 0.0
        acc_ref[2] = 0.0

    blk = out_ref[0]
    tgt = tgt_ref[0]
    y = tgt[:, 0:1]
    st = tgt[:, 1:2].astype(jnp.int32)

    col = jax.lax.broadcasted_iota(jnp.int32, (TBLK, OUT_CH), 1)
    is_sym = col >= 1

    neg_inf = jnp.float32(-jnp.inf)
    sym = jnp.where(is_sym, blk, neg_inf)
    m = jnp.max(sym, axis=1, keepdims=True)
    s = jnp.sum(jnp.exp(sym - m), axis=1, keepdims=True)
    logz = m + jnp.log(s)

    ll = jnp.sum(jnp.where(col == st + 1, blk, 0.0), axis=1, keepdims=True)
    ce = logz - ll

    x = blk[:, 0:1]
    bce = jnp.maximum(x, 0.0) - x * y + jnp.log1p(jnp.exp(-jnp.abs(x)))

    acc_ref[0] += jnp.sum(bce)
    acc_ref[1] += jnp.sum(ce * y)
    acc_ref[2] += jnp.sum(y)

    @pl.when(i == GRID - 1)
    def _final():
        count = acc_ref[2]
        symbol_loss = jnp.where(
            count != 0.0, acc_ref[1] / jnp.maximum(count, 1.0), 0.0
        )
        res_ref[0, 0] = acc_ref[0] / T + symbol_loss


def kernel(outputs, targets):
    res = pl.pallas_call(
        _loss_body,
        grid=(GRID,),
        in_specs=[
            pl.BlockSpec((1, TBLK, OUT_CH), lambda i: (0, i, 0)),
            pl.BlockSpec((1, TBLK, 2), lambda i: (0, i, 0)),
        ],
        out_specs=pl.BlockSpec((1, 1), lambda i: (0, 0), memory_space=pltpu.SMEM),
        out_shape=jax.ShapeDtypeStruct((1, 1), jnp.float32),
        scratch_shapes=[pltpu.SMEM((3,), jnp.float32)],
        compiler_params=pltpu.CompilerParams(
            dimension_semantics=("arbitrary",),
        ),
    )(outputs, targets)
    return res[0, 0]


# speedup vs baseline: 1.6235x; 1.1318x over previous
import jax
import jax.numpy as jnp
from jax.experimental import pallas as pl
from jax.experimental.pallas import tpu as pltpu

OUT_CH = 257
T = 8192
TL = 1024
GRID = T // TL


def _loss_body(out_ref, tgt_ref, res_ref, acc_ref):
    i = pl.program_id(0)

    @pl.when(i == 0)
    def _init():
        acc_ref[0] = 0.0
        acc_ref[1] = 0.0
        acc_ref[2] = 0.0

    data = out_ref[...]
    y = tgt_ref[0:1, :]
    st = tgt_ref[1:2, :].astype(jnp.int32)

    m = jnp.max(data, axis=0, keepdims=True)
    s_all = jnp.sum(jnp.exp(data - m), axis=0, keepdims=True)
    x = data[0:1, :]
    s_sym = s_all - jnp.exp(x - m)
    logz = m + jnp.log(s_sym)

    ch = jax.lax.broadcasted_iota(jnp.int32, (OUT_CH, TL), 0)
    ll = jnp.sum(jnp.where(ch == st + 1, data, 0.0), axis=0, keepdims=True)
    ce = logz - ll

    bce = jnp.maximum(x, 0.0) - x * y + jnp.log1p(jnp.exp(-jnp.abs(x)))

    acc_ref[0] += jnp.sum(bce)
    acc_ref[1] += jnp.sum(ce * y)
    acc_ref[2] += jnp.sum(y)

    @pl.when(i == GRID - 1)
    def _final():
        count = acc_ref[2]
        symbol_loss = jnp.where(
            count != 0.0, acc_ref[1] / jnp.maximum(count, 1.0), 0.0
        )
        res_ref[0, 0] = acc_ref[0] / T + symbol_loss


def kernel(outputs, targets):
    ot = jnp.transpose(outputs.reshape(T, OUT_CH))
    tt = jnp.transpose(targets.reshape(T, 2))
    res = pl.pallas_call(
        _loss_body,
        grid=(GRID,),
        in_specs=[
            pl.BlockSpec((OUT_CH, TL), lambda i: (0, i)),
            pl.BlockSpec((2, TL), lambda i: (0, i)),
        ],
        out_specs=pl.BlockSpec((1, 1), lambda i: (0, 0), memory_space=pltpu.SMEM),
        out_shape=jax.ShapeDtypeStruct((1, 1), jnp.float32),
        scratch_shapes=[pltpu.SMEM((3,), jnp.float32)],
        compiler_params=pltpu.CompilerParams(
            dimension_semantics=("arbitrary",),
        ),
    )(ot, tt)
    return res[0, 0]


# speedup vs baseline: 4.3146x; 2.6577x over previous
import jax
import jax.numpy as jnp
from jax.experimental import pallas as pl
from jax.experimental.pallas import tpu as pltpu

OUT_CH = 257
T = 8192
TL = 1024
GRID = T // TL


def _loss_body(out_ref, tgt_ref, res_ref, acc_ref):
    i = pl.program_id(0)

    @pl.when(i == 0)
    def _init():
        acc_ref[0] = 0.0
        acc_ref[1] = 0.0
        acc_ref[2] = 0.0

    data = out_ref[:, 0, :]
    y = tgt_ref[0:1, 0, :]
    st = tgt_ref[1:2, 0, :].astype(jnp.int32)

    m = jnp.max(data, axis=0, keepdims=True)
    s_all = jnp.sum(jnp.exp(data - m), axis=0, keepdims=True)
    x = data[0:1, :]
    s_sym = s_all - jnp.exp(x - m)
    logz = m + jnp.log(s_sym)

    ch = jax.lax.broadcasted_iota(jnp.int32, (OUT_CH, TL), 0)
    ll = jnp.sum(jnp.where(ch == st + 1, data, 0.0), axis=0, keepdims=True)
    ce = logz - ll

    bce = jnp.maximum(x, 0.0) - x * y + jnp.log1p(jnp.exp(-jnp.abs(x)))

    acc_ref[0] += jnp.sum(bce)
    acc_ref[1] += jnp.sum(ce * y)
    acc_ref[2] += jnp.sum(y)

    @pl.when(i == GRID - 1)
    def _final():
        count = acc_ref[2]
        symbol_loss = jnp.where(
            count != 0.0, acc_ref[1] / jnp.maximum(count, 1.0), 0.0
        )
        res_ref[0, 0] = acc_ref[0] / T + symbol_loss


def kernel(outputs, targets):
    ot = jnp.transpose(outputs, (2, 0, 1))
    tt = jnp.transpose(targets, (2, 0, 1))
    res = pl.pallas_call(
        _loss_body,
        grid=(GRID,),
        in_specs=[
            pl.BlockSpec((OUT_CH, 1, TL), lambda i: (0, 0, i)),
            pl.BlockSpec((2, 1, TL), lambda i: (0, 0, i)),
        ],
        out_specs=pl.BlockSpec((1, 1), lambda i: (0, 0), memory_space=pltpu.SMEM),
        out_shape=jax.ShapeDtypeStruct((1, 1), jnp.float32),
        scratch_shapes=[pltpu.SMEM((3,), jnp.float32)],
        compiler_params=pltpu.CompilerParams(
            dimension_semantics=("arbitrary",),
        ),
    )(ot, tt)
    return res[0, 0]
